# baseline (speedup 1.0000x reference)
import functools

import jax
import jax.numpy as jnp
from jax import lax
from jax.experimental import pallas as pl
from jax.experimental.pallas import tpu as pltpu
from jax.experimental.pallas import tpu_sc as plsc

N = 10000
L = 4
IN_C = 128
HID = 300
HIDP = 320
OUT_C = 128
E = 320000
BATCH = 128
E_PAD = 327680
EB = 2560
NB_DEG = 80
BG = 64
NB_SCAT = 320
CHB = 64
NCH = NB_SCAT // CHB
NPAD_D = 10240
RPT_D = 640
NPAD_S = 10112
RPT_S = 632
TRASH = 10000
TRASH_ROWS = 112
T1 = HIDP // 2
T2 = OUT_C // 2
NBLK = 1000
GRID_N = N // NBLK

_HIGH = lax.Precision.HIGHEST


def _h1_body(x_ref, w_ref, b_ref, o_ref):
    xm = jnp.mean(x_ref[...], axis=1)
    o_ref[...] = (
        lax.dot_general(xm, w_ref[...], (((1,), (0,)), ((), ())),
                        precision=_HIGH)
        + b_ref[...]
    )


def _edge_body(e_ref, srcp_ref, dstp_ref, goff_ref):
    i = pl.program_id(0)
    s = e_ref[0]
    d = e_ref[1]
    m = s == d
    base = (lax.broadcasted_iota(jnp.int32, (8, BATCH), 0) * BATCH
            + lax.broadcasted_iota(jnp.int32, (8, BATCH), 1) + i * (8 * BATCH))
    trash = TRASH + base % TRASH_ROWS
    srcp_ref[...] = jnp.where(m, trash, s)
    dstp_ref[...] = jnp.where(m, trash, d)
    g = jnp.where(m, (base * 9) % N, s)
    goff_ref[0] = g
    goff_ref[1] = g + N


def _dinv(d_ref):
    return lax.rsqrt(d_ref[0, :, 0] + d_ref[1, :, 0] + 1.0)


def _hs1_body(d_ref, h_ref, o_ref):
    dinv = _dinv(d_ref)
    hs = h_ref[...] * dinv[:, None]
    o_ref[0] = hs[:, :T1]
    o_ref[1] = hs[:, T1:]


def _mid_body(d_ref, acc_ref, hs_ref, w_ref, b_ref, o_ref):
    dinv = _dinv(d_ref)
    t = acc_ref[...] + hs_ref[...]
    p = jnp.concatenate([t[0], t[1]], axis=1) * dinv[:, None]
    p = jnp.where(p >= 0, p, 0.01 * p)
    h2 = (
        lax.dot_general(p, w_ref[...], (((1,), (0,)), ((), ())),
                        precision=_HIGH)
        + b_ref[...]
    )
    hs2 = h2 * dinv[:, None]
    o_ref[0] = hs2[:, :T2]
    o_ref[1] = hs2[:, T2:]


def _out_body(d_ref, acc_ref, hs_ref, o_ref):
    dinv = _dinv(d_ref)
    t = acc_ref[...] + hs_ref[...]
    o_ref[...] = jnp.concatenate([t[0], t[1]], axis=1) * dinv[:, None]


_MESH = plsc.VectorSubcoreMesh(core_axis_name="c", subcore_axis_name="s")
_SC_PARAMS = pltpu.CompilerParams(use_tc_tiling_on_sc=False)


@functools.partial(
    pl.kernel,
    mesh=_MESH,
    out_type=jax.ShapeDtypeStruct((2, NPAD_D), jnp.float32),
    compiler_params=_SC_PARAMS,
    scratch_types=[
        pltpu.VMEM((NB_DEG, BATCH), jnp.int32),
        pltpu.VMEM((BATCH,), jnp.float32),
        pltpu.VMEM((RPT_D,), jnp.float32),
        pltpu.VMEM_SHARED((NPAD_D,), jnp.float32),
    ],
)
def _sc_deg(srcp_hbm, ones_hbm, z640_hbm, deg_hbm, idxv, ones, obuf, degS):
    c = lax.axis_index("c")
    s = lax.axis_index("s")
    pltpu.sync_copy(ones_hbm, ones)
    pltpu.sync_copy(z640_hbm, obuf)
    pltpu.sync_copy(obuf, degS.at[pl.ds(s * RPT_D, RPT_D)])
    pltpu.sync_copy(srcp_hbm.at[c, s], idxv)
    plsc.subcore_barrier()

    @pl.loop(0, NB_DEG)
    def _(j):
        pltpu.sync_copy(ones, degS.at[idxv.at[j]], add=True)

    plsc.subcore_barrier()
    pltpu.sync_copy(degS.at[pl.ds(s * RPT_D, RPT_D)], obuf)
    pltpu.sync_copy(obuf, deg_hbm.at[c, pl.ds(s * RPT_D, RPT_D)])


def _make_sc_scat(T):
    @functools.partial(
        pl.kernel,
        mesh=_MESH,
        out_type=jax.ShapeDtypeStruct((2, NPAD_S, T), jnp.float32),
        compiler_params=_SC_PARAMS,
        scratch_types=[
            pltpu.VMEM((CHB, BG), jnp.int32),
            pltpu.VMEM((CHB, BG), jnp.int32),
            pltpu.VMEM((BG, T), jnp.float32),
            pltpu.VMEM((BG, T), jnp.float32),
            pltpu.VMEM_SHARED((NPAD_S, T), jnp.float32),
            pltpu.SemaphoreType.DMA,
        ],
    )
    def _scat(hst_hbm, goff_hbm, dstp_hbm, zrows_hbm, acc_hbm,
              srcv, dstv, g0, g1, accS, sem):
        c = lax.axis_index("c")
        s = lax.axis_index("s")
        pltpu.sync_copy(zrows_hbm, g0)

        @pl.loop(0, 9)
        def _(k):
            pltpu.sync_copy(g0, accS.at[pl.ds(s * RPT_S + k * BG, BG)])

        pltpu.sync_copy(g0.at[pl.ds(0, RPT_S - 9 * BG)],
                        accS.at[pl.ds(s * RPT_S + 9 * BG, RPT_S - 9 * BG)])
        plsc.subcore_barrier()

        @pl.loop(0, NCH)
        def _(q):
            pltpu.sync_copy(goff_hbm.at[c, s, pl.ds(q * CHB, CHB)], srcv)
            pltpu.sync_copy(dstp_hbm.at[s, pl.ds(q * CHB, CHB)], dstv)
            pltpu.async_copy(hst_hbm.at[srcv.at[0]], g0, sem)

            @pl.loop(0, CHB // 2 - 1)
            def _(p):
                j = 2 * p
                pltpu.make_async_copy(hst_hbm.at[srcv.at[j]], g0, sem).wait()
                pltpu.async_copy(hst_hbm.at[srcv.at[j + 1]], g1, sem)
                pltpu.sync_copy(g0, accS.at[dstv.at[j]], add=True)
                pltpu.make_async_copy(
                    hst_hbm.at[srcv.at[j + 1]], g1, sem).wait()
                pltpu.async_copy(hst_hbm.at[srcv.at[j + 2]], g0, sem)
                pltpu.sync_copy(g1, accS.at[dstv.at[j + 1]], add=True)

            pltpu.make_async_copy(
                hst_hbm.at[srcv.at[CHB - 2]], g0, sem).wait()
            pltpu.async_copy(hst_hbm.at[srcv.at[CHB - 1]], g1, sem)
            pltpu.sync_copy(g0, accS.at[dstv.at[CHB - 2]], add=True)
            pltpu.make_async_copy(
                hst_hbm.at[srcv.at[CHB - 1]], g1, sem).wait()
            pltpu.sync_copy(g1, accS.at[dstv.at[CHB - 1]], add=True)

        plsc.subcore_barrier()

        @pl.loop(0, 9)
        def _(k):
            pltpu.sync_copy(accS.at[pl.ds(s * RPT_S + k * BG, BG)], g0)
            pltpu.sync_copy(g0, acc_hbm.at[c, pl.ds(s * RPT_S + k * BG, BG)])

        pltpu.sync_copy(accS.at[pl.ds(s * RPT_S + 9 * BG, RPT_S - 9 * BG)],
                        g0.at[pl.ds(0, RPT_S - 9 * BG)])
        pltpu.sync_copy(g0.at[pl.ds(0, RPT_S - 9 * BG)],
                        acc_hbm.at[c, pl.ds(s * RPT_S + 9 * BG,
                                            RPT_S - 9 * BG)])

    return _scat


_sc_scat1 = _make_sc_scat(T1)
_sc_scat2 = _make_sc_scat(T2)


def kernel(x, edge_index, W1, b1, W2, b2):
    f32 = jnp.float32
    W1p = jnp.pad(W1, ((0, 0), (0, HIDP - HID)))
    b1p = jnp.pad(b1, (0, HIDP - HID)).reshape(1, HIDP)
    W2p = jnp.pad(W2, ((0, HIDP - HID), (0, 0)))
    b2r = b2.reshape(1, OUT_C)
    ei3 = jnp.pad(edge_index, ((0, 0), (0, E_PAD - E))).reshape(2, EB, BATCH)

    ones128 = jnp.ones((BATCH,), f32)
    z640 = jnp.zeros((RPT_D,), f32)
    z1 = jnp.zeros((BG, T1), f32)
    z2 = jnp.zeros((BG, T2), f32)

    h1 = pl.pallas_call(
        _h1_body,
        grid=(GRID_N,),
        in_specs=[
            pl.BlockSpec((NBLK, L, IN_C), lambda i: (i, 0, 0)),
            pl.BlockSpec((IN_C, HIDP), lambda i: (0, 0)),
            pl.BlockSpec((1, HIDP), lambda i: (0, 0)),
        ],
        out_specs=pl.BlockSpec((NBLK, HIDP), lambda i: (i, 0)),
        out_shape=jax.ShapeDtypeStruct((N, HIDP), f32),
    )(x, W1p, b1p)

    srcp, dstp, goff = pl.pallas_call(
        _edge_body,
        grid=(EB // 8,),
        in_specs=[pl.BlockSpec((2, 8, BATCH), lambda i: (0, i, 0))],
        out_specs=[
            pl.BlockSpec((8, BATCH), lambda i: (i, 0)),
            pl.BlockSpec((8, BATCH), lambda i: (i, 0)),
            pl.BlockSpec((2, 8, BATCH), lambda i: (0, i, 0)),
        ],
        out_shape=[
            jax.ShapeDtypeStruct((EB, BATCH), jnp.int32),
            jax.ShapeDtypeStruct((EB, BATCH), jnp.int32),
            jax.ShapeDtypeStruct((2, EB, BATCH), jnp.int32),
        ],
    )(ei3)

    srcp_r = srcp.reshape(2, 16, NB_DEG, BATCH)
    dstp_r = dstp.reshape(16, NB_SCAT, BG)
    goff_r = goff.reshape(2, 16, NB_SCAT, BG)

    deg2 = _sc_deg(srcp_r, ones128, z640).reshape(2, NPAD_D, 1)

    hst1 = pl.pallas_call(
        _hs1_body,
        grid=(GRID_N,),
        in_specs=[
            pl.BlockSpec((2, NBLK, 1), lambda i: (0, i, 0)),
            pl.BlockSpec((NBLK, HIDP), lambda i: (i, 0)),
        ],
        out_specs=pl.BlockSpec((2, NBLK, T1), lambda i: (0, i, 0)),
        out_shape=jax.ShapeDtypeStruct((2, N, T1), f32),
    )(deg2, h1)

    acc1 = _sc_scat1(hst1.reshape(2 * N, T1), goff_r, dstp_r, z1)

    hst2 = pl.pallas_call(
        _mid_body,
        grid=(GRID_N,),
        in_specs=[
            pl.BlockSpec((2, NBLK, 1), lambda i: (0, i, 0)),
            pl.BlockSpec((2, NBLK, T1), lambda i: (0, i, 0)),
            pl.BlockSpec((2, NBLK, T1), lambda i: (0, i, 0)),
            pl.BlockSpec((HIDP, OUT_C), lambda i: (0, 0)),
            pl.BlockSpec((1, OUT_C), lambda i: (0, 0)),
        ],
        out_specs=pl.BlockSpec((2, NBLK, T2), lambda i: (0, i, 0)),
        out_shape=jax.ShapeDtypeStruct((2, N, T2), f32),
    )(deg2, acc1, hst1, W2p, b2r)

    acc2 = _sc_scat2(hst2.reshape(2 * N, T2), goff_r, dstp_r, z2)

    out = pl.pallas_call(
        _out_body,
        grid=(GRID_N,),
        in_specs=[
            pl.BlockSpec((2, NBLK, 1), lambda i: (0, i, 0)),
            pl.BlockSpec((2, NBLK, T2), lambda i: (0, i, 0)),
            pl.BlockSpec((2, NBLK, T2), lambda i: (0, i, 0)),
        ],
        out_specs=pl.BlockSpec((NBLK, OUT_C), lambda i: (i, 0)),
        out_shape=jax.ShapeDtypeStruct((N, OUT_C), f32),
    )(deg2, acc2, hst2)

    return out


# speedup vs baseline: 1.2141x; 1.2141x over previous
import functools

import jax
import jax.numpy as jnp
from jax import lax
from jax.experimental import pallas as pl
from jax.experimental.pallas import tpu as pltpu
from jax.experimental.pallas import tpu_sc as plsc

N = 10000
L = 4
IN_C = 128
HID = 300
HIDP = 320
OUT_C = 128
E = 320000
BATCH = 128
E_PAD = 327680
EB = 2560
NB_DEG = 80
BG = 64
NB_SCAT = 320
CHB = 64
NCH = NB_SCAT // CHB
NPAD_D = 10240
RPT_D = 640
NPAD_S = 10112
RPT_S = 632
TRASH = 10000
TRASH_ROWS = 112
T1 = HIDP // 2
T2 = OUT_C // 2
NBLK = 1000
GRID_N = N // NBLK

_HIGH = lax.Precision.HIGHEST


def _h1_body(x_ref, w_ref, b_ref, o_ref):
    xm = jnp.mean(x_ref[...], axis=1)
    o_ref[...] = (
        lax.dot_general(xm, w_ref[...], (((1,), (0,)), ((), ())),
                        precision=_HIGH)
        + b_ref[...]
    )


EBLK = 128


def _edge_body(e_ref, srcp_ref, dstp_ref, goff_ref):
    i = pl.program_id(0)
    s = e_ref[0]
    d = e_ref[1]
    m = s == d
    base = (lax.broadcasted_iota(jnp.int32, (EBLK, BATCH), 0) * BATCH
            + lax.broadcasted_iota(jnp.int32, (EBLK, BATCH), 1)
            + i * (EBLK * BATCH))
    trash = TRASH + base % TRASH_ROWS
    srcp_ref[...] = jnp.where(m, trash, s)
    dstp_ref[...] = jnp.where(m, trash, d)
    g = jnp.where(m, (base * 9) % N, s)
    goff_ref[0] = g
    goff_ref[1] = g + N


def _dinv(d_ref):
    return lax.rsqrt(d_ref[0, :, 0] + d_ref[1, :, 0] + 1.0)


def _hs1_body(d_ref, h_ref, o_ref):
    dinv = _dinv(d_ref)
    hs = h_ref[...] * dinv[:, None]
    o_ref[0] = hs[:, :T1]
    o_ref[1] = hs[:, T1:]


def _mid_body(d_ref, acc_ref, hs_ref, w_ref, b_ref, o_ref):
    dinv = _dinv(d_ref)
    t = acc_ref[...] + hs_ref[...]
    p = jnp.concatenate([t[0], t[1]], axis=1) * dinv[:, None]
    p = jnp.where(p >= 0, p, 0.01 * p)
    h2 = (
        lax.dot_general(p, w_ref[...], (((1,), (0,)), ((), ())),
                        precision=_HIGH)
        + b_ref[...]
    )
    hs2 = h2 * dinv[:, None]
    o_ref[0] = hs2[:, :T2]
    o_ref[1] = hs2[:, T2:]


def _out_body(d_ref, acc_ref, hs_ref, o_ref):
    dinv = _dinv(d_ref)
    t = acc_ref[...] + hs_ref[...]
    o_ref[...] = jnp.concatenate([t[0], t[1]], axis=1) * dinv[:, None]


_MESH = plsc.VectorSubcoreMesh(core_axis_name="c", subcore_axis_name="s")
_SC_PARAMS = pltpu.CompilerParams(use_tc_tiling_on_sc=False)


@functools.partial(
    pl.kernel,
    mesh=_MESH,
    out_type=jax.ShapeDtypeStruct((2, NPAD_D), jnp.float32),
    compiler_params=_SC_PARAMS,
    scratch_types=[
        pltpu.VMEM((NB_DEG, BATCH), jnp.int32),
        pltpu.VMEM((BATCH,), jnp.float32),
        pltpu.VMEM((RPT_D,), jnp.float32),
        pltpu.VMEM_SHARED((NPAD_D,), jnp.float32),
    ],
)
def _sc_deg(srcp_hbm, ones_hbm, z640_hbm, deg_hbm, idxv, ones, obuf, degS):
    c = lax.axis_index("c")
    s = lax.axis_index("s")
    pltpu.sync_copy(ones_hbm, ones)
    pltpu.sync_copy(z640_hbm, obuf)
    pltpu.sync_copy(obuf, degS.at[pl.ds(s * RPT_D, RPT_D)])
    pltpu.sync_copy(srcp_hbm.at[c, s], idxv)
    plsc.subcore_barrier()

    @pl.loop(0, NB_DEG)
    def _(j):
        pltpu.sync_copy(ones, degS.at[idxv.at[j]], add=True)

    plsc.subcore_barrier()
    pltpu.sync_copy(degS.at[pl.ds(s * RPT_D, RPT_D)], obuf)
    pltpu.sync_copy(obuf, deg_hbm.at[c, pl.ds(s * RPT_D, RPT_D)])


def _make_sc_scat(T, bg, chb, dbuf):
    nb = E_PAD // 16 // bg
    nch = nb // chb
    nfull = RPT_S // bg
    rem = RPT_S - nfull * bg
    scratch = [
        pltpu.VMEM((chb, bg), jnp.int32),
        pltpu.VMEM((chb, bg), jnp.int32),
        pltpu.VMEM((bg, T), jnp.float32),
        pltpu.VMEM((bg, T), jnp.float32) if dbuf else None,
        pltpu.VMEM_SHARED((NPAD_S, T), jnp.float32),
        pltpu.SemaphoreType.DMA,
    ]
    scratch = [sc for sc in scratch if sc is not None]

    def _body(hst_hbm, goff_hbm, dstp_hbm, zrows_hbm, acc_hbm,
              srcv, dstv, g0, *rest):
        if dbuf:
            g1, accS, sem = rest
        else:
            accS, sem = rest
        c = lax.axis_index("c")
        s = lax.axis_index("s")
        pltpu.sync_copy(zrows_hbm, g0)

        @pl.loop(0, nfull)
        def _(k):
            pltpu.sync_copy(g0, accS.at[pl.ds(s * RPT_S + k * bg, bg)])

        if rem:
            pltpu.sync_copy(g0.at[pl.ds(0, rem)],
                            accS.at[pl.ds(s * RPT_S + nfull * bg, rem)])
        plsc.subcore_barrier()

        @pl.loop(0, nch)
        def _(q):
            pltpu.sync_copy(goff_hbm.at[c, s, pl.ds(q * chb, chb)], srcv)
            pltpu.sync_copy(dstp_hbm.at[s, pl.ds(q * chb, chb)], dstv)
            if not dbuf:
                @pl.loop(0, chb)
                def _(j):
                    pltpu.sync_copy(hst_hbm.at[srcv.at[j]], g0)
                    pltpu.sync_copy(g0, accS.at[dstv.at[j]], add=True)
            else:
                pltpu.async_copy(hst_hbm.at[srcv.at[0]], g0, sem)

                @pl.loop(0, chb // 2 - 1)
                def _(p):
                    j = 2 * p
                    pltpu.make_async_copy(
                        hst_hbm.at[srcv.at[j]], g0, sem).wait()
                    pltpu.async_copy(hst_hbm.at[srcv.at[j + 1]], g1, sem)
                    pltpu.sync_copy(g0, accS.at[dstv.at[j]], add=True)
                    pltpu.make_async_copy(
                        hst_hbm.at[srcv.at[j + 1]], g1, sem).wait()
                    pltpu.async_copy(hst_hbm.at[srcv.at[j + 2]], g0, sem)
                    pltpu.sync_copy(g1, accS.at[dstv.at[j + 1]], add=True)

                pltpu.make_async_copy(
                    hst_hbm.at[srcv.at[chb - 2]], g0, sem).wait()
                pltpu.async_copy(hst_hbm.at[srcv.at[chb - 1]], g1, sem)
                pltpu.sync_copy(g0, accS.at[dstv.at[chb - 2]], add=True)
                pltpu.make_async_copy(
                    hst_hbm.at[srcv.at[chb - 1]], g1, sem).wait()
                pltpu.sync_copy(g1, accS.at[dstv.at[chb - 1]], add=True)

        plsc.subcore_barrier()

        @pl.loop(0, nfull)
        def _(k):
            pltpu.sync_copy(accS.at[pl.ds(s * RPT_S + k * bg, bg)], g0)
            pltpu.sync_copy(g0, acc_hbm.at[c, pl.ds(s * RPT_S + k * bg, bg)])

        if rem:
            pltpu.sync_copy(accS.at[pl.ds(s * RPT_S + nfull * bg, rem)],
                            g0.at[pl.ds(0, rem)])
            pltpu.sync_copy(g0.at[pl.ds(0, rem)],
                            acc_hbm.at[c, pl.ds(s * RPT_S + nfull * bg, rem)])

    return pl.kernel(
        _body,
        mesh=_MESH,
        out_type=jax.ShapeDtypeStruct((2, NPAD_S, T), jnp.float32),
        compiler_params=_SC_PARAMS,
        scratch_types=scratch,
    )


BG1, CHB1 = 64, 64
BG2, CHB2 = 128, 32
_sc_scat1 = _make_sc_scat(T1, BG1, CHB1, True)
_sc_scat2 = _make_sc_scat(T2, BG2, CHB2, False)


def kernel(x, edge_index, W1, b1, W2, b2):
    f32 = jnp.float32
    W1p = jnp.pad(W1, ((0, 0), (0, HIDP - HID)))
    b1p = jnp.pad(b1, (0, HIDP - HID)).reshape(1, HIDP)
    W2p = jnp.pad(W2, ((0, HIDP - HID), (0, 0)))
    b2r = b2.reshape(1, OUT_C)
    ei3 = jnp.pad(edge_index, ((0, 0), (0, E_PAD - E))).reshape(2, EB, BATCH)

    ones128 = jnp.ones((BATCH,), f32)
    z640 = jnp.zeros((RPT_D,), f32)
    z1 = jnp.zeros((BG1, T1), f32)
    z2 = jnp.zeros((BG2, T2), f32)

    h1 = pl.pallas_call(
        _h1_body,
        grid=(GRID_N,),
        in_specs=[
            pl.BlockSpec((NBLK, L, IN_C), lambda i: (i, 0, 0)),
            pl.BlockSpec((IN_C, HIDP), lambda i: (0, 0)),
            pl.BlockSpec((1, HIDP), lambda i: (0, 0)),
        ],
        out_specs=pl.BlockSpec((NBLK, HIDP), lambda i: (i, 0)),
        out_shape=jax.ShapeDtypeStruct((N, HIDP), f32),
    )(x, W1p, b1p)

    srcp, dstp, goff = pl.pallas_call(
        _edge_body,
        grid=(EB // EBLK,),
        in_specs=[pl.BlockSpec((2, EBLK, BATCH), lambda i: (0, i, 0))],
        out_specs=[
            pl.BlockSpec((EBLK, BATCH), lambda i: (i, 0)),
            pl.BlockSpec((EBLK, BATCH), lambda i: (i, 0)),
            pl.BlockSpec((2, EBLK, BATCH), lambda i: (0, i, 0)),
        ],
        out_shape=[
            jax.ShapeDtypeStruct((EB, BATCH), jnp.int32),
            jax.ShapeDtypeStruct((EB, BATCH), jnp.int32),
            jax.ShapeDtypeStruct((2, EB, BATCH), jnp.int32),
        ],
    )(ei3)

    srcp_r = srcp.reshape(2, 16, NB_DEG, BATCH)
    dstp_r1 = dstp.reshape(16, E_PAD // 16 // BG1, BG1)
    goff_r1 = goff.reshape(2, 16, E_PAD // 16 // BG1, BG1)
    dstp_r2 = dstp.reshape(16, E_PAD // 16 // BG2, BG2)
    goff_r2 = goff.reshape(2, 16, E_PAD // 16 // BG2, BG2)

    deg2 = _sc_deg(srcp_r, ones128, z640).reshape(2, NPAD_D, 1)

    hst1 = pl.pallas_call(
        _hs1_body,
        grid=(GRID_N,),
        in_specs=[
            pl.BlockSpec((2, NBLK, 1), lambda i: (0, i, 0)),
            pl.BlockSpec((NBLK, HIDP), lambda i: (i, 0)),
        ],
        out_specs=pl.BlockSpec((2, NBLK, T1), lambda i: (0, i, 0)),
        out_shape=jax.ShapeDtypeStruct((2, N, T1), f32),
    )(deg2, h1)

    acc1 = _sc_scat1(hst1.reshape(2 * N, T1), goff_r1, dstp_r1, z1)

    hst2 = pl.pallas_call(
        _mid_body,
        grid=(GRID_N,),
        in_specs=[
            pl.BlockSpec((2, NBLK, 1), lambda i: (0, i, 0)),
            pl.BlockSpec((2, NBLK, T1), lambda i: (0, i, 0)),
            pl.BlockSpec((2, NBLK, T1), lambda i: (0, i, 0)),
            pl.BlockSpec((HIDP, OUT_C), lambda i: (0, 0)),
            pl.BlockSpec((1, OUT_C), lambda i: (0, 0)),
        ],
        out_specs=pl.BlockSpec((2, NBLK, T2), lambda i: (0, i, 0)),
        out_shape=jax.ShapeDtypeStruct((2, N, T2), f32),
    )(deg2, acc1, hst1, W2p, b2r)

    acc2 = _sc_scat2(hst2.reshape(2 * N, T2), goff_r2, dstp_r2, z2)

    out = pl.pallas_call(
        _out_body,
        grid=(GRID_N,),
        in_specs=[
            pl.BlockSpec((2, NBLK, 1), lambda i: (0, i, 0)),
            pl.BlockSpec((2, NBLK, T2), lambda i: (0, i, 0)),
            pl.BlockSpec((2, NBLK, T2), lambda i: (0, i, 0)),
        ],
        out_specs=pl.BlockSpec((NBLK, OUT_C), lambda i: (i, 0)),
        out_shape=jax.ShapeDtypeStruct((N, OUT_C), f32),
    )(deg2, acc2, hst2)

    return out


# speedup vs baseline: 1.3030x; 1.0732x over previous
import functools

import jax
import jax.numpy as jnp
from jax import lax
from jax.experimental import pallas as pl
from jax.experimental.pallas import tpu as pltpu
from jax.experimental.pallas import tpu_sc as plsc

N = 10000
L = 4
IN_C = 128
HID = 300
HIDP = 320
OUT_C = 128
E = 320000
BATCH = 128
E_PAD = 327680
EB = 2560
NB_DEG = 80
BG = 64
NB_SCAT = 320
CHB = 64
NCH = NB_SCAT // CHB
NPAD_D = 10240
RPT_D = 640
NPAD_S = 10112
RPT_S = 632
TRASH = 10000
TRASH_ROWS = 112
T1 = HIDP // 2
T2 = OUT_C // 2
NBLK = 1000
GRID_N = N // NBLK

_HIGH = lax.Precision.HIGHEST


def _h1_body(x_ref, w_ref, b_ref, o_ref):
    xm = jnp.mean(x_ref[...], axis=1)
    o_ref[...] = (
        lax.dot_general(xm, w_ref[...], (((1,), (0,)), ((), ())),
                        precision=_HIGH)
        + b_ref[...]
    )


EBLK = 128


def _edge_body(e_ref, srcp_ref, dstp_ref, goff_ref):
    i = pl.program_id(0)
    s = e_ref[0]
    d = e_ref[1]
    m = s == d
    base = (lax.broadcasted_iota(jnp.int32, (EBLK, BATCH), 0) * BATCH
            + lax.broadcasted_iota(jnp.int32, (EBLK, BATCH), 1)
            + i * (EBLK * BATCH))
    trash = TRASH + base % TRASH_ROWS
    srcp_ref[...] = jnp.where(m, trash, s)
    dstp_ref[...] = jnp.where(m, trash, d)
    g = jnp.where(m, (base * 9) % N, s)
    goff_ref[0] = g
    goff_ref[1] = g + N


def _dinv(d_ref):
    return lax.rsqrt(d_ref[0, :, 0] + d_ref[1, :, 0] + 1.0)


def _hs1_body(d_ref, h_ref, o_ref):
    dinv = _dinv(d_ref)
    hs = h_ref[...] * dinv[:, None]
    o_ref[0] = hs[:, :T1]
    o_ref[1] = hs[:, T1:]


def _mid_body(d_ref, acc_ref, hs_ref, w_ref, b_ref, o_ref):
    dinv = _dinv(d_ref)
    t = acc_ref[...] + hs_ref[...]
    p = jnp.concatenate([t[0], t[1]], axis=1) * dinv[:, None]
    p = jnp.where(p >= 0, p, 0.01 * p)
    h2 = (
        lax.dot_general(p, w_ref[...], (((1,), (0,)), ((), ())),
                        precision=_HIGH)
        + b_ref[...]
    )
    hs2 = h2 * dinv[:, None]
    o_ref[0] = hs2[:, :T2]
    o_ref[1] = hs2[:, T2:]


def _out_body(d_ref, acc_ref, hs_ref, o_ref):
    dinv = _dinv(d_ref)
    t = acc_ref[...] + hs_ref[...]
    o_ref[...] = jnp.concatenate([t[0], t[1]], axis=1) * dinv[:, None]


_MESH = plsc.VectorSubcoreMesh(core_axis_name="c", subcore_axis_name="s")
_SC_PARAMS = pltpu.CompilerParams(use_tc_tiling_on_sc=False)


@functools.partial(
    pl.kernel,
    mesh=_MESH,
    out_type=jax.ShapeDtypeStruct((2, NPAD_D), jnp.float32),
    compiler_params=_SC_PARAMS,
    scratch_types=[
        pltpu.VMEM((NB_DEG, BATCH), jnp.int32),
        pltpu.VMEM((BATCH,), jnp.float32),
        pltpu.VMEM((RPT_D,), jnp.float32),
        pltpu.VMEM_SHARED((NPAD_D,), jnp.float32),
    ],
)
def _sc_deg(srcp_hbm, ones_hbm, z640_hbm, deg_hbm, idxv, ones, obuf, degS):
    c = lax.axis_index("c")
    s = lax.axis_index("s")
    pltpu.sync_copy(ones_hbm, ones)
    pltpu.sync_copy(z640_hbm, obuf)
    pltpu.sync_copy(obuf, degS.at[pl.ds(s * RPT_D, RPT_D)])
    pltpu.sync_copy(srcp_hbm.at[c, s], idxv)
    plsc.subcore_barrier()

    @pl.loop(0, NB_DEG)
    def _(j):
        pltpu.sync_copy(ones, degS.at[idxv.at[j]], add=True)

    plsc.subcore_barrier()
    pltpu.sync_copy(degS.at[pl.ds(s * RPT_D, RPT_D)], obuf)
    pltpu.sync_copy(obuf, deg_hbm.at[c, pl.ds(s * RPT_D, RPT_D)])


def _make_sc_scat(T, bg, chb, dbuf):
    nb = E_PAD // 16 // bg
    nch = nb // chb
    nfull = RPT_S // bg
    rem = RPT_S - nfull * bg
    scratch = [
        pltpu.VMEM((chb, bg), jnp.int32),
        pltpu.VMEM((chb, bg), jnp.int32),
        pltpu.VMEM((bg, T), jnp.float32),
        pltpu.VMEM((bg, T), jnp.float32) if dbuf else None,
        pltpu.VMEM_SHARED((NPAD_S, T), jnp.float32),
        pltpu.SemaphoreType.DMA,
    ]
    scratch = [sc for sc in scratch if sc is not None]

    def _body(hst_hbm, goff_hbm, dstp_hbm, zrows_hbm, acc_hbm,
              srcv, dstv, g0, *rest):
        if dbuf:
            g1, accS, sem = rest
        else:
            accS, sem = rest
        c = lax.axis_index("c")
        s = lax.axis_index("s")
        pltpu.sync_copy(zrows_hbm, g0)

        @pl.loop(0, nfull)
        def _(k):
            pltpu.sync_copy(g0, accS.at[pl.ds(s * RPT_S + k * bg, bg)])

        if rem:
            pltpu.sync_copy(g0.at[pl.ds(0, rem)],
                            accS.at[pl.ds(s * RPT_S + nfull * bg, rem)])
        plsc.subcore_barrier()

        @pl.loop(0, nch)
        def _(q):
            pltpu.sync_copy(goff_hbm.at[c, s, pl.ds(q * chb, chb)], srcv)
            pltpu.sync_copy(dstp_hbm.at[s, pl.ds(q * chb, chb)], dstv)
            if not dbuf:
                @pl.loop(0, chb)
                def _(j):
                    pltpu.sync_copy(hst_hbm.at[srcv.at[j]], g0)
                    pltpu.sync_copy(g0, accS.at[dstv.at[j]], add=True)
            else:
                pltpu.async_copy(hst_hbm.at[srcv.at[0]], g0, sem)

                @pl.loop(0, chb // 2 - 1)
                def _(p):
                    j = 2 * p
                    pltpu.make_async_copy(
                        hst_hbm.at[srcv.at[j]], g0, sem).wait()
                    pltpu.async_copy(hst_hbm.at[srcv.at[j + 1]], g1, sem)
                    pltpu.sync_copy(g0, accS.at[dstv.at[j]], add=True)
                    pltpu.make_async_copy(
                        hst_hbm.at[srcv.at[j + 1]], g1, sem).wait()
                    pltpu.async_copy(hst_hbm.at[srcv.at[j + 2]], g0, sem)
                    pltpu.sync_copy(g1, accS.at[dstv.at[j + 1]], add=True)

                pltpu.make_async_copy(
                    hst_hbm.at[srcv.at[chb - 2]], g0, sem).wait()
                pltpu.async_copy(hst_hbm.at[srcv.at[chb - 1]], g1, sem)
                pltpu.sync_copy(g0, accS.at[dstv.at[chb - 2]], add=True)
                pltpu.make_async_copy(
                    hst_hbm.at[srcv.at[chb - 1]], g1, sem).wait()
                pltpu.sync_copy(g1, accS.at[dstv.at[chb - 1]], add=True)

        plsc.subcore_barrier()

        @pl.loop(0, nfull)
        def _(k):
            pltpu.sync_copy(accS.at[pl.ds(s * RPT_S + k * bg, bg)], g0)
            pltpu.sync_copy(g0, acc_hbm.at[c, pl.ds(s * RPT_S + k * bg, bg)])

        if rem:
            pltpu.sync_copy(accS.at[pl.ds(s * RPT_S + nfull * bg, rem)],
                            g0.at[pl.ds(0, rem)])
            pltpu.sync_copy(g0.at[pl.ds(0, rem)],
                            acc_hbm.at[c, pl.ds(s * RPT_S + nfull * bg, rem)])

    return pl.kernel(
        _body,
        mesh=_MESH,
        out_type=jax.ShapeDtypeStruct((2, NPAD_S, T), jnp.float32),
        compiler_params=_SC_PARAMS,
        scratch_types=scratch,
    )


BG1, CHB1 = 64, 64
BG2, CHB2 = 128, 32
_sc_scat1 = _make_sc_scat(T1, BG1, CHB1, True)
_sc_scat2 = _make_sc_scat(T2, BG2, CHB2, True)


def kernel(x, edge_index, W1, b1, W2, b2):
    f32 = jnp.float32
    W1p = jnp.pad(W1, ((0, 0), (0, HIDP - HID)))
    b1p = jnp.pad(b1, (0, HIDP - HID)).reshape(1, HIDP)
    W2p = jnp.pad(W2, ((0, HIDP - HID), (0, 0)))
    b2r = b2.reshape(1, OUT_C)
    ei3 = jnp.pad(edge_index, ((0, 0), (0, E_PAD - E))).reshape(2, EB, BATCH)

    ones128 = jnp.ones((BATCH,), f32)
    z640 = jnp.zeros((RPT_D,), f32)
    z1 = jnp.zeros((BG1, T1), f32)
    z2 = jnp.zeros((BG2, T2), f32)

    h1 = pl.pallas_call(
        _h1_body,
        grid=(GRID_N,),
        in_specs=[
            pl.BlockSpec((NBLK, L, IN_C), lambda i: (i, 0, 0)),
            pl.BlockSpec((IN_C, HIDP), lambda i: (0, 0)),
            pl.BlockSpec((1, HIDP), lambda i: (0, 0)),
        ],
        out_specs=pl.BlockSpec((NBLK, HIDP), lambda i: (i, 0)),
        out_shape=jax.ShapeDtypeStruct((N, HIDP), f32),
    )(x, W1p, b1p)

    srcp, dstp, goff = pl.pallas_call(
        _edge_body,
        grid=(EB // EBLK,),
        in_specs=[pl.BlockSpec((2, EBLK, BATCH), lambda i: (0, i, 0))],
        out_specs=[
            pl.BlockSpec((EBLK, BATCH), lambda i: (i, 0)),
            pl.BlockSpec((EBLK, BATCH), lambda i: (i, 0)),
            pl.BlockSpec((2, EBLK, BATCH), lambda i: (0, i, 0)),
        ],
        out_shape=[
            jax.ShapeDtypeStruct((EB, BATCH), jnp.int32),
            jax.ShapeDtypeStruct((EB, BATCH), jnp.int32),
            jax.ShapeDtypeStruct((2, EB, BATCH), jnp.int32),
        ],
    )(ei3)

    srcp_r = srcp.reshape(2, 16, NB_DEG, BATCH)
    dstp_r1 = dstp.reshape(16, E_PAD // 16 // BG1, BG1)
    goff_r1 = goff.reshape(2, 16, E_PAD // 16 // BG1, BG1)
    dstp_r2 = dstp.reshape(16, E_PAD // 16 // BG2, BG2)
    goff_r2 = goff.reshape(2, 16, E_PAD // 16 // BG2, BG2)

    deg2 = _sc_deg(srcp_r, ones128, z640).reshape(2, NPAD_D, 1)

    hst1 = pl.pallas_call(
        _hs1_body,
        grid=(GRID_N,),
        in_specs=[
            pl.BlockSpec((2, NBLK, 1), lambda i: (0, i, 0)),
            pl.BlockSpec((NBLK, HIDP), lambda i: (i, 0)),
        ],
        out_specs=pl.BlockSpec((2, NBLK, T1), lambda i: (0, i, 0)),
        out_shape=jax.ShapeDtypeStruct((2, N, T1), f32),
    )(deg2, h1)

    acc1 = _sc_scat1(hst1.reshape(2 * N, T1), goff_r1, dstp_r1, z1)

    hst2 = pl.pallas_call(
        _mid_body,
        grid=(GRID_N,),
        in_specs=[
            pl.BlockSpec((2, NBLK, 1), lambda i: (0, i, 0)),
            pl.BlockSpec((2, NBLK, T1), lambda i: (0, i, 0)),
            pl.BlockSpec((2, NBLK, T1), lambda i: (0, i, 0)),
            pl.BlockSpec((HIDP, OUT_C), lambda i: (0, 0)),
            pl.BlockSpec((1, OUT_C), lambda i: (0, 0)),
        ],
        out_specs=pl.BlockSpec((2, NBLK, T2), lambda i: (0, i, 0)),
        out_shape=jax.ShapeDtypeStruct((2, N, T2), f32),
    )(deg2, acc1, hst1, W2p, b2r)

    acc2 = _sc_scat2(hst2.reshape(2 * N, T2), goff_r2, dstp_r2, z2)

    out = pl.pallas_call(
        _out_body,
        grid=(GRID_N,),
        in_specs=[
            pl.BlockSpec((2, NBLK, 1), lambda i: (0, i, 0)),
            pl.BlockSpec((2, NBLK, T2), lambda i: (0, i, 0)),
            pl.BlockSpec((2, NBLK, T2), lambda i: (0, i, 0)),
        ],
        out_specs=pl.BlockSpec((NBLK, OUT_C), lambda i: (i, 0)),
        out_shape=jax.ShapeDtypeStruct((N, OUT_C), f32),
    )(deg2, acc2, hst2)

    return out


# speedup vs baseline: 1.3163x; 1.0102x over previous
import functools

import jax
import jax.numpy as jnp
from jax import lax
from jax.experimental import pallas as pl
from jax.experimental.pallas import tpu as pltpu
from jax.experimental.pallas import tpu_sc as plsc

N = 10000
L = 4
IN_C = 128
HID = 300
HIDP = 320
OUT_C = 128
E = 320000
BATCH = 128
E_PAD = 327680
EB = 2560
NB_DEG = 80
BG = 64
NB_SCAT = 320
CHB = 64
NCH = NB_SCAT // CHB
NPAD_D = 10240
RPT_D = 640
NPAD_S = 10112
RPT_S = 632
TRASH = 10000
TRASH_ROWS = 112
T1 = HIDP // 2
T2 = OUT_C // 2
NBLK = 1000
GRID_N = N // NBLK

_HIGH = lax.Precision.HIGHEST


def _h1_body(x_ref, w_ref, b_ref, o_ref):
    xm = jnp.mean(x_ref[...], axis=1)
    o_ref[...] = (
        lax.dot_general(xm, w_ref[...], (((1,), (0,)), ((), ())),
                        precision=_HIGH)
        + b_ref[...]
    )


EBLK = 128


def _edge_body(e_ref, srcp_ref, dstp_ref, goff_ref):
    i = pl.program_id(0)
    s = e_ref[0]
    d = e_ref[1]
    m = s == d
    base = (lax.broadcasted_iota(jnp.int32, (EBLK, BATCH), 0) * BATCH
            + lax.broadcasted_iota(jnp.int32, (EBLK, BATCH), 1)
            + i * (EBLK * BATCH))
    trash = TRASH + base % TRASH_ROWS
    srcp_ref[...] = jnp.where(m, trash, s)
    dstp_ref[...] = jnp.where(m, trash, d)
    g = jnp.where(m, (base * 9) % N, s)
    goff_ref[0] = g
    goff_ref[1] = g + N


def _dinv(d_ref):
    return lax.rsqrt(d_ref[0, :, 0] + d_ref[1, :, 0] + 1.0)


def _hs1_body(d_ref, h_ref, o_ref):
    dinv = _dinv(d_ref)
    hs = h_ref[...] * dinv[:, None]
    o_ref[0] = hs[:, :T1]
    o_ref[1] = hs[:, T1:]


def _mid_body(d_ref, acc_ref, hs_ref, w_ref, b_ref, o_ref):
    dinv = _dinv(d_ref)
    t = acc_ref[...] + hs_ref[...]
    p = jnp.concatenate([t[0], t[1]], axis=1) * dinv[:, None]
    p = jnp.where(p >= 0, p, 0.01 * p)
    h2 = (
        lax.dot_general(p, w_ref[...], (((1,), (0,)), ((), ())),
                        precision=_HIGH)
        + b_ref[...]
    )
    hs2 = h2 * dinv[:, None]
    o_ref[0] = hs2[:, :T2]
    o_ref[1] = hs2[:, T2:]


def _out_body(d_ref, acc_ref, hs_ref, o_ref):
    dinv = _dinv(d_ref)
    t = acc_ref[...] + hs_ref[...]
    o_ref[...] = jnp.concatenate([t[0], t[1]], axis=1) * dinv[:, None]


_MESH = plsc.VectorSubcoreMesh(core_axis_name="c", subcore_axis_name="s")
_SC_PARAMS = pltpu.CompilerParams(use_tc_tiling_on_sc=False)


@functools.partial(
    pl.kernel,
    mesh=_MESH,
    out_type=jax.ShapeDtypeStruct((2, NPAD_D), jnp.float32),
    compiler_params=_SC_PARAMS,
    scratch_types=[
        pltpu.VMEM((NB_DEG, BATCH), jnp.int32),
        pltpu.VMEM((BATCH,), jnp.float32),
        pltpu.VMEM((RPT_D,), jnp.float32),
        pltpu.VMEM_SHARED((NPAD_D,), jnp.float32),
    ],
)
def _sc_deg(srcp_hbm, ones_hbm, z640_hbm, deg_hbm, idxv, ones, obuf, degS):
    c = lax.axis_index("c")
    s = lax.axis_index("s")
    pltpu.sync_copy(ones_hbm, ones)
    pltpu.sync_copy(z640_hbm, obuf)
    pltpu.sync_copy(obuf, degS.at[pl.ds(s * RPT_D, RPT_D)])
    pltpu.sync_copy(srcp_hbm.at[c, s], idxv)
    plsc.subcore_barrier()

    @pl.loop(0, NB_DEG)
    def _(j):
        pltpu.sync_copy(ones, degS.at[idxv.at[j]], add=True)

    plsc.subcore_barrier()
    pltpu.sync_copy(degS.at[pl.ds(s * RPT_D, RPT_D)], obuf)
    pltpu.sync_copy(obuf, deg_hbm.at[c, pl.ds(s * RPT_D, RPT_D)])


def _make_sc_scat(T, bg, chb, dbuf):
    nb = E_PAD // 16 // bg
    nch = nb // chb
    assert nch % 2 == 0
    nfull = RPT_S // bg
    rem = RPT_S - nfull * bg
    scratch = [
        pltpu.VMEM((chb, bg), jnp.int32),
        pltpu.VMEM((chb, bg), jnp.int32),
        pltpu.VMEM((chb, bg), jnp.int32),
        pltpu.VMEM((chb, bg), jnp.int32),
        pltpu.VMEM((bg, T), jnp.float32),
        pltpu.VMEM((bg, T), jnp.float32),
        pltpu.VMEM_SHARED((NPAD_S, T), jnp.float32),
        pltpu.SemaphoreType.DMA,
        pltpu.SemaphoreType.DMA,
    ]

    def _body(hst_hbm, goff_hbm, dstp_hbm, zrows_hbm, acc_hbm,
              srcA, dstA, srcB, dstB, g0, g1, accS, sem, semi):
        c = lax.axis_index("c")
        s = lax.axis_index("s")
        pltpu.sync_copy(zrows_hbm, g0)

        @pl.loop(0, nfull)
        def _(k):
            pltpu.sync_copy(g0, accS.at[pl.ds(s * RPT_S + k * bg, bg)])

        if rem:
            pltpu.sync_copy(g0.at[pl.ds(0, rem)],
                            accS.at[pl.ds(s * RPT_S + nfull * bg, rem)])
        plsc.subcore_barrier()

        def _process(srcv, dstv):
            pltpu.async_copy(hst_hbm.at[srcv.at[0]], g0, sem)

            @pl.loop(0, chb // 2 - 1)
            def _(p):
                j = 2 * p
                pltpu.make_async_copy(hst_hbm.at[srcv.at[j]], g0, sem).wait()
                pltpu.async_copy(hst_hbm.at[srcv.at[j + 1]], g1, sem)
                pltpu.sync_copy(g0, accS.at[dstv.at[j]], add=True)
                pltpu.make_async_copy(
                    hst_hbm.at[srcv.at[j + 1]], g1, sem).wait()
                pltpu.async_copy(hst_hbm.at[srcv.at[j + 2]], g0, sem)
                pltpu.sync_copy(g1, accS.at[dstv.at[j + 1]], add=True)

            pltpu.make_async_copy(hst_hbm.at[srcv.at[chb - 2]], g0, sem).wait()
            pltpu.async_copy(hst_hbm.at[srcv.at[chb - 1]], g1, sem)
            pltpu.sync_copy(g0, accS.at[dstv.at[chb - 2]], add=True)
            pltpu.make_async_copy(hst_hbm.at[srcv.at[chb - 1]], g1, sem).wait()
            pltpu.sync_copy(g1, accS.at[dstv.at[chb - 1]], add=True)

        def _load_idx(q, sv, dv):
            pltpu.async_copy(goff_hbm.at[c, s, pl.ds(q * chb, chb)], sv, semi)
            pltpu.async_copy(dstp_hbm.at[s, pl.ds(q * chb, chb)], dv, semi)

        def _wait_idx(q, sv, dv):
            pltpu.make_async_copy(
                goff_hbm.at[c, s, pl.ds(q * chb, chb)], sv, semi).wait()
            pltpu.make_async_copy(
                dstp_hbm.at[s, pl.ds(q * chb, chb)], dv, semi).wait()

        pltpu.sync_copy(goff_hbm.at[c, s, pl.ds(0, chb)], srcA)
        pltpu.sync_copy(dstp_hbm.at[s, pl.ds(0, chb)], dstA)

        @pl.loop(0, nch // 2)
        def _(u):
            _load_idx(2 * u + 1, srcB, dstB)
            _process(srcA, dstA)
            _wait_idx(2 * u + 1, srcB, dstB)

            @pl.when(u < nch // 2 - 1)
            def _():
                _load_idx(2 * u + 2, srcA, dstA)

            _process(srcB, dstB)

            @pl.when(u < nch // 2 - 1)
            def _():
                _wait_idx(2 * u + 2, srcA, dstA)

        plsc.subcore_barrier()
        pltpu.sync_copy(accS.at[pl.ds(s * RPT_S, RPT_S)],
                        acc_hbm.at[c, pl.ds(s * RPT_S, RPT_S)])

    return pl.kernel(
        _body,
        mesh=_MESH,
        out_type=jax.ShapeDtypeStruct((2, NPAD_S, T), jnp.float32),
        compiler_params=_SC_PARAMS,
        scratch_types=scratch,
    )


BG1, CHB1 = 64, 32
BG2, CHB2 = 128, 16
_sc_scat1 = _make_sc_scat(T1, BG1, CHB1, True)
_sc_scat2 = _make_sc_scat(T2, BG2, CHB2, True)


def kernel(x, edge_index, W1, b1, W2, b2):
    f32 = jnp.float32
    W1p = jnp.pad(W1, ((0, 0), (0, HIDP - HID)))
    b1p = jnp.pad(b1, (0, HIDP - HID)).reshape(1, HIDP)
    W2p = jnp.pad(W2, ((0, HIDP - HID), (0, 0)))
    b2r = b2.reshape(1, OUT_C)
    ei3 = jnp.pad(edge_index, ((0, 0), (0, E_PAD - E))).reshape(2, EB, BATCH)

    ones128 = jnp.ones((BATCH,), f32)
    z640 = jnp.zeros((RPT_D,), f32)
    z1 = jnp.zeros((BG1, T1), f32)
    z2 = jnp.zeros((BG2, T2), f32)

    h1 = pl.pallas_call(
        _h1_body,
        grid=(GRID_N,),
        in_specs=[
            pl.BlockSpec((NBLK, L, IN_C), lambda i: (i, 0, 0)),
            pl.BlockSpec((IN_C, HIDP), lambda i: (0, 0)),
            pl.BlockSpec((1, HIDP), lambda i: (0, 0)),
        ],
        out_specs=pl.BlockSpec((NBLK, HIDP), lambda i: (i, 0)),
        out_shape=jax.ShapeDtypeStruct((N, HIDP), f32),
    )(x, W1p, b1p)

    srcp, dstp, goff = pl.pallas_call(
        _edge_body,
        grid=(EB // EBLK,),
        in_specs=[pl.BlockSpec((2, EBLK, BATCH), lambda i: (0, i, 0))],
        out_specs=[
            pl.BlockSpec((EBLK, BATCH), lambda i: (i, 0)),
            pl.BlockSpec((EBLK, BATCH), lambda i: (i, 0)),
            pl.BlockSpec((2, EBLK, BATCH), lambda i: (0, i, 0)),
        ],
        out_shape=[
            jax.ShapeDtypeStruct((EB, BATCH), jnp.int32),
            jax.ShapeDtypeStruct((EB, BATCH), jnp.int32),
            jax.ShapeDtypeStruct((2, EB, BATCH), jnp.int32),
        ],
    )(ei3)

    srcp_r = srcp.reshape(2, 16, NB_DEG, BATCH)
    dstp_r1 = dstp.reshape(16, E_PAD // 16 // BG1, BG1)
    goff_r1 = goff.reshape(2, 16, E_PAD // 16 // BG1, BG1)
    dstp_r2 = dstp.reshape(16, E_PAD // 16 // BG2, BG2)
    goff_r2 = goff.reshape(2, 16, E_PAD // 16 // BG2, BG2)

    deg2 = _sc_deg(srcp_r, ones128, z640).reshape(2, NPAD_D, 1)

    hst1 = pl.pallas_call(
        _hs1_body,
        grid=(GRID_N,),
        in_specs=[
            pl.BlockSpec((2, NBLK, 1), lambda i: (0, i, 0)),
            pl.BlockSpec((NBLK, HIDP), lambda i: (i, 0)),
        ],
        out_specs=pl.BlockSpec((2, NBLK, T1), lambda i: (0, i, 0)),
        out_shape=jax.ShapeDtypeStruct((2, N, T1), f32),
    )(deg2, h1)

    acc1 = _sc_scat1(hst1.reshape(2 * N, T1), goff_r1, dstp_r1, z1)

    hst2 = pl.pallas_call(
        _mid_body,
        grid=(GRID_N,),
        in_specs=[
            pl.BlockSpec((2, NBLK, 1), lambda i: (0, i, 0)),
            pl.BlockSpec((2, NBLK, T1), lambda i: (0, i, 0)),
            pl.BlockSpec((2, NBLK, T1), lambda i: (0, i, 0)),
            pl.BlockSpec((HIDP, OUT_C), lambda i: (0, 0)),
            pl.BlockSpec((1, OUT_C), lambda i: (0, 0)),
        ],
        out_specs=pl.BlockSpec((2, NBLK, T2), lambda i: (0, i, 0)),
        out_shape=jax.ShapeDtypeStruct((2, N, T2), f32),
    )(deg2, acc1, hst1, W2p, b2r)

    acc2 = _sc_scat2(hst2.reshape(2 * N, T2), goff_r2, dstp_r2, z2)

    out = pl.pallas_call(
        _out_body,
        grid=(GRID_N,),
        in_specs=[
            pl.BlockSpec((2, NBLK, 1), lambda i: (0, i, 0)),
            pl.BlockSpec((2, NBLK, T2), lambda i: (0, i, 0)),
            pl.BlockSpec((2, NBLK, T2), lambda i: (0, i, 0)),
        ],
        out_specs=pl.BlockSpec((NBLK, OUT_C), lambda i: (i, 0)),
        out_shape=jax.ShapeDtypeStruct((N, OUT_C), f32),
    )(deg2, acc2, hst2)

    return out


# speedup vs baseline: 1.3606x; 1.0336x over previous
import functools

import jax
import jax.numpy as jnp
from jax import lax
from jax.experimental import pallas as pl
from jax.experimental.pallas import tpu as pltpu
from jax.experimental.pallas import tpu_sc as plsc

N = 10000
L = 4
IN_C = 128
HID = 300
HIDP = 304
OUT_C = 128
E = 320000
BATCH = 128
E_PAD = 327680
EB = 2560
NB_DEG = 80
BG = 64
NB_SCAT = 320
CHB = 64
NCH = NB_SCAT // CHB
NPAD_D = 10240
RPT_D = 640
NPAD_S = 10112
RPT_S = 632
TRASH = 10000
TRASH_ROWS = 112
T1 = HIDP // 2
T2 = OUT_C // 2
NBLK = 1000
GRID_N = N // NBLK

_HIGH = lax.Precision.HIGHEST


def _h1_body(x_ref, w_ref, b_ref, o_ref):
    xm = jnp.mean(x_ref[...], axis=1)
    o_ref[...] = (
        lax.dot_general(xm, w_ref[...], (((1,), (0,)), ((), ())),
                        precision=_HIGH)
        + b_ref[...]
    )


EBLK = 128


def _edge_body(e_ref, srcp_ref, dstp_ref, goff_ref):
    i = pl.program_id(0)
    s = e_ref[0]
    d = e_ref[1]
    m = s == d
    base = (lax.broadcasted_iota(jnp.int32, (EBLK, BATCH), 0) * BATCH
            + lax.broadcasted_iota(jnp.int32, (EBLK, BATCH), 1)
            + i * (EBLK * BATCH))
    trash = TRASH + base % TRASH_ROWS
    srcp_ref[...] = jnp.where(m, trash, s)
    dstp_ref[...] = jnp.where(m, trash, d)
    g = jnp.where(m, (base * 9) % N, s)
    goff_ref[0] = g
    goff_ref[1] = g + N


def _dinv(d_ref):
    return lax.rsqrt(d_ref[0, :, 0] + d_ref[1, :, 0] + 1.0)


def _hs1_body(d_ref, h_ref, o_ref):
    dinv = _dinv(d_ref)
    hs = h_ref[...] * dinv[:, None]
    o_ref[0] = hs[:, :T1]
    o_ref[1] = hs[:, T1:]


def _mid_body(d_ref, acc_ref, hs_ref, w_ref, b_ref, o_ref):
    dinv = _dinv(d_ref)
    t = acc_ref[...] + hs_ref[...]
    p = jnp.concatenate([t[0], t[1]], axis=1) * dinv[:, None]
    p = jnp.where(p >= 0, p, 0.01 * p)
    h2 = (
        lax.dot_general(p, w_ref[...], (((1,), (0,)), ((), ())),
                        precision=_HIGH)
        + b_ref[...]
    )
    hs2 = h2 * dinv[:, None]
    o_ref[0] = hs2[:, :T2]
    o_ref[1] = hs2[:, T2:]


def _out_body(d_ref, acc_ref, hs_ref, o_ref):
    dinv = _dinv(d_ref)
    t = acc_ref[...] + hs_ref[...]
    o_ref[...] = jnp.concatenate([t[0], t[1]], axis=1) * dinv[:, None]


_MESH = plsc.VectorSubcoreMesh(core_axis_name="c", subcore_axis_name="s")
_SC_PARAMS = pltpu.CompilerParams(use_tc_tiling_on_sc=False)


@functools.partial(
    pl.kernel,
    mesh=_MESH,
    out_type=jax.ShapeDtypeStruct((2, NPAD_D), jnp.float32),
    compiler_params=_SC_PARAMS,
    scratch_types=[
        pltpu.VMEM((NB_DEG, BATCH), jnp.int32),
        pltpu.VMEM((BATCH,), jnp.float32),
        pltpu.VMEM((RPT_D,), jnp.float32),
        pltpu.VMEM_SHARED((NPAD_D,), jnp.float32),
    ],
)
def _sc_deg(srcp_hbm, ones_hbm, z640_hbm, deg_hbm, idxv, ones, obuf, degS):
    c = lax.axis_index("c")
    s = lax.axis_index("s")
    pltpu.sync_copy(ones_hbm, ones)
    pltpu.sync_copy(z640_hbm, obuf)
    pltpu.sync_copy(obuf, degS.at[pl.ds(s * RPT_D, RPT_D)])
    pltpu.sync_copy(srcp_hbm.at[c, s], idxv)
    plsc.subcore_barrier()

    @pl.loop(0, NB_DEG)
    def _(j):
        pltpu.sync_copy(ones, degS.at[idxv.at[j]], add=True)

    plsc.subcore_barrier()
    pltpu.sync_copy(degS.at[pl.ds(s * RPT_D, RPT_D)], obuf)
    pltpu.sync_copy(obuf, deg_hbm.at[c, pl.ds(s * RPT_D, RPT_D)])


def _make_sc_scat(T, bg, chb, dbuf):
    nb = E_PAD // 16 // bg
    nch = nb // chb
    assert nch % 2 == 0
    nfull = RPT_S // bg
    rem = RPT_S - nfull * bg
    scratch = [
        pltpu.VMEM((chb, bg), jnp.int32),
        pltpu.VMEM((chb, bg), jnp.int32),
        pltpu.VMEM((chb, bg), jnp.int32),
        pltpu.VMEM((chb, bg), jnp.int32),
        pltpu.VMEM((bg, T), jnp.float32),
        pltpu.VMEM((bg, T), jnp.float32),
        pltpu.VMEM_SHARED((NPAD_S, T), jnp.float32),
        pltpu.SemaphoreType.DMA,
        pltpu.SemaphoreType.DMA,
    ]

    def _body(hst_hbm, goff_hbm, dstp_hbm, zrows_hbm, acc_hbm,
              srcA, dstA, srcB, dstB, g0, g1, accS, sem, semi):
        c = lax.axis_index("c")
        s = lax.axis_index("s")
        pltpu.sync_copy(zrows_hbm, g0)

        @pl.loop(0, nfull)
        def _(k):
            pltpu.sync_copy(g0, accS.at[pl.ds(s * RPT_S + k * bg, bg)])

        if rem:
            pltpu.sync_copy(g0.at[pl.ds(0, rem)],
                            accS.at[pl.ds(s * RPT_S + nfull * bg, rem)])
        plsc.subcore_barrier()

        def _process(srcv, dstv):
            pltpu.async_copy(hst_hbm.at[srcv.at[0]], g0, sem)

            @pl.loop(0, chb // 2 - 1)
            def _(p):
                j = 2 * p
                pltpu.make_async_copy(hst_hbm.at[srcv.at[j]], g0, sem).wait()
                pltpu.async_copy(hst_hbm.at[srcv.at[j + 1]], g1, sem)
                pltpu.sync_copy(g0, accS.at[dstv.at[j]], add=True)
                pltpu.make_async_copy(
                    hst_hbm.at[srcv.at[j + 1]], g1, sem).wait()
                pltpu.async_copy(hst_hbm.at[srcv.at[j + 2]], g0, sem)
                pltpu.sync_copy(g1, accS.at[dstv.at[j + 1]], add=True)

            pltpu.make_async_copy(hst_hbm.at[srcv.at[chb - 2]], g0, sem).wait()
            pltpu.async_copy(hst_hbm.at[srcv.at[chb - 1]], g1, sem)
            pltpu.sync_copy(g0, accS.at[dstv.at[chb - 2]], add=True)
            pltpu.make_async_copy(hst_hbm.at[srcv.at[chb - 1]], g1, sem).wait()
            pltpu.sync_copy(g1, accS.at[dstv.at[chb - 1]], add=True)

        def _load_idx(q, sv, dv):
            pltpu.async_copy(goff_hbm.at[c, s, pl.ds(q * chb, chb)], sv, semi)
            pltpu.async_copy(dstp_hbm.at[s, pl.ds(q * chb, chb)], dv, semi)

        def _wait_idx(q, sv, dv):
            pltpu.make_async_copy(
                goff_hbm.at[c, s, pl.ds(q * chb, chb)], sv, semi).wait()
            pltpu.make_async_copy(
                dstp_hbm.at[s, pl.ds(q * chb, chb)], dv, semi).wait()

        pltpu.sync_copy(goff_hbm.at[c, s, pl.ds(0, chb)], srcA)
        pltpu.sync_copy(dstp_hbm.at[s, pl.ds(0, chb)], dstA)

        @pl.loop(0, nch // 2)
        def _(u):
            _load_idx(2 * u + 1, srcB, dstB)
            _process(srcA, dstA)
            _wait_idx(2 * u + 1, srcB, dstB)

            @pl.when(u < nch // 2 - 1)
            def _():
                _load_idx(2 * u + 2, srcA, dstA)

            _process(srcB, dstB)

            @pl.when(u < nch // 2 - 1)
            def _():
                _wait_idx(2 * u + 2, srcA, dstA)

        plsc.subcore_barrier()
        pltpu.sync_copy(accS.at[pl.ds(s * RPT_S, RPT_S)],
                        acc_hbm.at[c, pl.ds(s * RPT_S, RPT_S)])

    return pl.kernel(
        _body,
        mesh=_MESH,
        out_type=jax.ShapeDtypeStruct((2, NPAD_S, T), jnp.float32),
        compiler_params=_SC_PARAMS,
        scratch_types=scratch,
    )


BG1, CHB1 = 80, 32
BG2, CHB2 = 128, 16
_sc_scat1 = _make_sc_scat(T1, BG1, CHB1, True)
_sc_scat2 = _make_sc_scat(T2, BG2, CHB2, True)


def kernel(x, edge_index, W1, b1, W2, b2):
    f32 = jnp.float32
    W1p = jnp.pad(W1, ((0, 0), (0, HIDP - HID)))
    b1p = jnp.pad(b1, (0, HIDP - HID)).reshape(1, HIDP)
    W2p = jnp.pad(W2, ((0, HIDP - HID), (0, 0)))
    b2r = b2.reshape(1, OUT_C)
    ei3 = jnp.pad(edge_index, ((0, 0), (0, E_PAD - E))).reshape(2, EB, BATCH)

    ones128 = jnp.ones((BATCH,), f32)
    z640 = jnp.zeros((RPT_D,), f32)
    z1 = jnp.zeros((BG1, T1), f32)
    z2 = jnp.zeros((BG2, T2), f32)

    h1 = pl.pallas_call(
        _h1_body,
        grid=(GRID_N,),
        in_specs=[
            pl.BlockSpec((NBLK, L, IN_C), lambda i: (i, 0, 0)),
            pl.BlockSpec((IN_C, HIDP), lambda i: (0, 0)),
            pl.BlockSpec((1, HIDP), lambda i: (0, 0)),
        ],
        out_specs=pl.BlockSpec((NBLK, HIDP), lambda i: (i, 0)),
        out_shape=jax.ShapeDtypeStruct((N, HIDP), f32),
    )(x, W1p, b1p)

    srcp, dstp, goff = pl.pallas_call(
        _edge_body,
        grid=(EB // EBLK,),
        in_specs=[pl.BlockSpec((2, EBLK, BATCH), lambda i: (0, i, 0))],
        out_specs=[
            pl.BlockSpec((EBLK, BATCH), lambda i: (i, 0)),
            pl.BlockSpec((EBLK, BATCH), lambda i: (i, 0)),
            pl.BlockSpec((2, EBLK, BATCH), lambda i: (0, i, 0)),
        ],
        out_shape=[
            jax.ShapeDtypeStruct((EB, BATCH), jnp.int32),
            jax.ShapeDtypeStruct((EB, BATCH), jnp.int32),
            jax.ShapeDtypeStruct((2, EB, BATCH), jnp.int32),
        ],
    )(ei3)

    srcp_r = srcp.reshape(2, 16, NB_DEG, BATCH)
    dstp_r1 = dstp.reshape(16, E_PAD // 16 // BG1, BG1)
    goff_r1 = goff.reshape(2, 16, E_PAD // 16 // BG1, BG1)
    dstp_r2 = dstp.reshape(16, E_PAD // 16 // BG2, BG2)
    goff_r2 = goff.reshape(2, 16, E_PAD // 16 // BG2, BG2)

    deg2 = _sc_deg(srcp_r, ones128, z640).reshape(2, NPAD_D, 1)

    hst1 = pl.pallas_call(
        _hs1_body,
        grid=(GRID_N,),
        in_specs=[
            pl.BlockSpec((2, NBLK, 1), lambda i: (0, i, 0)),
            pl.BlockSpec((NBLK, HIDP), lambda i: (i, 0)),
        ],
        out_specs=pl.BlockSpec((2, NBLK, T1), lambda i: (0, i, 0)),
        out_shape=jax.ShapeDtypeStruct((2, N, T1), f32),
    )(deg2, h1)

    acc1 = _sc_scat1(hst1.reshape(2 * N, T1), goff_r1, dstp_r1, z1)

    hst2 = pl.pallas_call(
        _mid_body,
        grid=(GRID_N,),
        in_specs=[
            pl.BlockSpec((2, NBLK, 1), lambda i: (0, i, 0)),
            pl.BlockSpec((2, NBLK, T1), lambda i: (0, i, 0)),
            pl.BlockSpec((2, NBLK, T1), lambda i: (0, i, 0)),
            pl.BlockSpec((HIDP, OUT_C), lambda i: (0, 0)),
            pl.BlockSpec((1, OUT_C), lambda i: (0, 0)),
        ],
        out_specs=pl.BlockSpec((2, NBLK, T2), lambda i: (0, i, 0)),
        out_shape=jax.ShapeDtypeStruct((2, N, T2), f32),
    )(deg2, acc1, hst1, W2p, b2r)

    acc2 = _sc_scat2(hst2.reshape(2 * N, T2), goff_r2, dstp_r2, z2)

    out = pl.pallas_call(
        _out_body,
        grid=(GRID_N,),
        in_specs=[
            pl.BlockSpec((2, NBLK, 1), lambda i: (0, i, 0)),
            pl.BlockSpec((2, NBLK, T2), lambda i: (0, i, 0)),
            pl.BlockSpec((2, NBLK, T2), lambda i: (0, i, 0)),
        ],
        out_specs=pl.BlockSpec((NBLK, OUT_C), lambda i: (i, 0)),
        out_shape=jax.ShapeDtypeStruct((N, OUT_C), f32),
    )(deg2, acc2, hst2)

    return out


# speedup vs baseline: 1.3667x; 1.0045x over previous
import functools

import jax
import jax.numpy as jnp
from jax import lax
from jax.experimental import pallas as pl
from jax.experimental.pallas import tpu as pltpu
from jax.experimental.pallas import tpu_sc as plsc

N = 10000
L = 4
IN_C = 128
HID = 300
HIDP = 304
OUT_C = 128
E = 320000
BATCH = 128
E_PAD = 327680
EB = 2560
NB_DEG = 80
BG = 64
NB_SCAT = 320
CHB = 64
NCH = NB_SCAT // CHB
NPAD_D = 10240
RPT_D = 640
NPAD_S = 10112
RPT_S = 632
TRASH = 10000
TRASH_ROWS = 112
T1 = HIDP // 2
T2 = OUT_C // 2
NBLK = 1000
GRID_N = N // NBLK

_HIGH = lax.Precision.HIGHEST


def _h1_body(x_ref, w_ref, b_ref, o_ref):
    xm = jnp.mean(x_ref[...], axis=1)
    o_ref[...] = (
        lax.dot_general(xm, w_ref[...], (((1,), (0,)), ((), ())),
                        precision=_HIGH)
        + b_ref[...]
    )


EBLK = 128


def _edge_body(e_ref, srcp_ref, dstp_ref, goff_ref):
    i = pl.program_id(0)
    s = e_ref[0]
    d = e_ref[1]
    m = s == d
    base = (lax.broadcasted_iota(jnp.int32, (EBLK, BATCH), 0) * BATCH
            + lax.broadcasted_iota(jnp.int32, (EBLK, BATCH), 1)
            + i * (EBLK * BATCH))
    trash = TRASH + base % TRASH_ROWS
    srcp_ref[...] = jnp.where(m, trash, s)
    dstp_ref[...] = jnp.where(m, trash, d)
    g = jnp.where(m, (base * 9) % N, s)
    goff_ref[0] = g
    goff_ref[1] = g + N


def _dinv(d_ref):
    return lax.rsqrt(d_ref[0, :, 0] + d_ref[1, :, 0] + 1.0)


def _hs1_body(d_ref, h_ref, o_ref):
    dinv = _dinv(d_ref)
    hs = h_ref[...] * dinv[:, None]
    o_ref[0] = hs[:, :T1]
    o_ref[1] = hs[:, T1:]


def _mid_body(d_ref, acc_ref, hs_ref, w_ref, b_ref, o_ref):
    dinv = _dinv(d_ref)
    t = acc_ref[...] + hs_ref[...]
    p = jnp.concatenate([t[0], t[1]], axis=1) * dinv[:, None]
    p = jnp.where(p >= 0, p, 0.01 * p)
    h2 = (
        lax.dot_general(p, w_ref[...], (((1,), (0,)), ((), ())),
                        precision=_HIGH)
        + b_ref[...]
    )
    hs2 = h2 * dinv[:, None]
    o_ref[0] = hs2[:, :T2]
    o_ref[1] = hs2[:, T2:]


def _out_body(d_ref, acc_ref, hs_ref, o_ref):
    dinv = _dinv(d_ref)
    t = acc_ref[...] + hs_ref[...]
    o_ref[...] = jnp.concatenate([t[0], t[1]], axis=1) * dinv[:, None]


_MESH = plsc.VectorSubcoreMesh(core_axis_name="c", subcore_axis_name="s")
_SC_PARAMS = pltpu.CompilerParams(use_tc_tiling_on_sc=False)


@functools.partial(
    pl.kernel,
    mesh=_MESH,
    out_type=jax.ShapeDtypeStruct((2, NPAD_D), jnp.float32),
    compiler_params=_SC_PARAMS,
    scratch_types=[
        pltpu.VMEM((NB_DEG, BATCH), jnp.int32),
        pltpu.VMEM((BATCH,), jnp.float32),
        pltpu.VMEM((RPT_D,), jnp.float32),
        pltpu.VMEM_SHARED((NPAD_D,), jnp.float32),
    ],
)
def _sc_deg(srcp_hbm, ones_hbm, z640_hbm, deg_hbm, idxv, ones, obuf, degS):
    c = lax.axis_index("c")
    s = lax.axis_index("s")
    pltpu.sync_copy(ones_hbm, ones)
    pltpu.sync_copy(z640_hbm, obuf)
    pltpu.sync_copy(obuf, degS.at[pl.ds(s * RPT_D, RPT_D)])
    pltpu.sync_copy(srcp_hbm.at[c, s], idxv)
    plsc.subcore_barrier()

    @pl.loop(0, NB_DEG)
    def _(j):
        pltpu.sync_copy(ones, degS.at[idxv.at[j]], add=True)

    plsc.subcore_barrier()
    pltpu.sync_copy(degS.at[pl.ds(s * RPT_D, RPT_D)], obuf)
    pltpu.sync_copy(obuf, deg_hbm.at[c, pl.ds(s * RPT_D, RPT_D)])


def _make_sc_scat(T, bg, chb, dbuf):
    nb = E_PAD // 16 // bg
    nch = nb // chb
    assert nch % 2 == 0
    nfull = RPT_S // bg
    rem = RPT_S - nfull * bg
    scratch = [
        pltpu.VMEM((chb, bg), jnp.int32),
        pltpu.VMEM((chb, bg), jnp.int32),
        pltpu.VMEM((chb, bg), jnp.int32),
        pltpu.VMEM((chb, bg), jnp.int32),
        pltpu.VMEM((bg, T), jnp.float32),
        pltpu.VMEM((bg, T), jnp.float32),
        pltpu.VMEM_SHARED((NPAD_S, T), jnp.float32),
        pltpu.SemaphoreType.DMA,
        pltpu.SemaphoreType.DMA,
    ]

    def _body(hst_hbm, goff_hbm, dstp_hbm, zrows_hbm, acc_hbm,
              srcA, dstA, srcB, dstB, g0, g1, accS, sem, semi):
        c = lax.axis_index("c")
        s = lax.axis_index("s")
        pltpu.sync_copy(zrows_hbm, g0)

        @pl.loop(0, nfull)
        def _(k):
            pltpu.sync_copy(g0, accS.at[pl.ds(s * RPT_S + k * bg, bg)])

        if rem:
            pltpu.sync_copy(g0.at[pl.ds(0, rem)],
                            accS.at[pl.ds(s * RPT_S + nfull * bg, rem)])
        plsc.subcore_barrier()

        def _process(srcv, dstv):
            pltpu.async_copy(hst_hbm.at[srcv.at[0]], g0, sem)

            @pl.loop(0, chb // 2 - 1)
            def _(p):
                j = 2 * p
                pltpu.make_async_copy(hst_hbm.at[srcv.at[j]], g0, sem).wait()
                pltpu.async_copy(hst_hbm.at[srcv.at[j + 1]], g1, sem)
                pltpu.sync_copy(g0, accS.at[dstv.at[j]], add=True)
                pltpu.make_async_copy(
                    hst_hbm.at[srcv.at[j + 1]], g1, sem).wait()
                pltpu.async_copy(hst_hbm.at[srcv.at[j + 2]], g0, sem)
                pltpu.sync_copy(g1, accS.at[dstv.at[j + 1]], add=True)

            pltpu.make_async_copy(hst_hbm.at[srcv.at[chb - 2]], g0, sem).wait()
            pltpu.async_copy(hst_hbm.at[srcv.at[chb - 1]], g1, sem)
            pltpu.sync_copy(g0, accS.at[dstv.at[chb - 2]], add=True)
            pltpu.make_async_copy(hst_hbm.at[srcv.at[chb - 1]], g1, sem).wait()
            pltpu.sync_copy(g1, accS.at[dstv.at[chb - 1]], add=True)

        def _load_idx(q, sv, dv):
            pltpu.async_copy(goff_hbm.at[c, s, pl.ds(q * chb, chb)], sv, semi)
            pltpu.async_copy(dstp_hbm.at[s, pl.ds(q * chb, chb)], dv, semi)

        def _wait_idx(q, sv, dv):
            pltpu.make_async_copy(
                goff_hbm.at[c, s, pl.ds(q * chb, chb)], sv, semi).wait()
            pltpu.make_async_copy(
                dstp_hbm.at[s, pl.ds(q * chb, chb)], dv, semi).wait()

        pltpu.sync_copy(goff_hbm.at[c, s, pl.ds(0, chb)], srcA)
        pltpu.sync_copy(dstp_hbm.at[s, pl.ds(0, chb)], dstA)

        @pl.loop(0, nch // 2)
        def _(u):
            _load_idx(2 * u + 1, srcB, dstB)
            _process(srcA, dstA)
            _wait_idx(2 * u + 1, srcB, dstB)

            @pl.when(u < nch // 2 - 1)
            def _():
                _load_idx(2 * u + 2, srcA, dstA)

            _process(srcB, dstB)

            @pl.when(u < nch // 2 - 1)
            def _():
                _wait_idx(2 * u + 2, srcA, dstA)

        plsc.subcore_barrier()
        pltpu.sync_copy(accS.at[pl.ds(s * RPT_S, RPT_S)],
                        acc_hbm.at[c, pl.ds(s * RPT_S, RPT_S)])

    return pl.kernel(
        _body,
        mesh=_MESH,
        out_type=jax.ShapeDtypeStruct((2, NPAD_S, T), jnp.float32),
        compiler_params=_SC_PARAMS,
        scratch_types=scratch,
    )


BG1, CHB1 = 80, 32
BG2, CHB2 = 128, 80
_sc_scat1 = _make_sc_scat(T1, BG1, CHB1, True)
_sc_scat2 = _make_sc_scat(T2, BG2, CHB2, True)


def kernel(x, edge_index, W1, b1, W2, b2):
    f32 = jnp.float32
    W1p = jnp.pad(W1, ((0, 0), (0, HIDP - HID)))
    b1p = jnp.pad(b1, (0, HIDP - HID)).reshape(1, HIDP)
    W2p = jnp.pad(W2, ((0, HIDP - HID), (0, 0)))
    b2r = b2.reshape(1, OUT_C)
    ei3 = jnp.pad(edge_index, ((0, 0), (0, E_PAD - E))).reshape(2, EB, BATCH)

    ones128 = jnp.ones((BATCH,), f32)
    z640 = jnp.zeros((RPT_D,), f32)
    z1 = jnp.zeros((BG1, T1), f32)
    z2 = jnp.zeros((BG2, T2), f32)

    h1 = pl.pallas_call(
        _h1_body,
        grid=(GRID_N,),
        in_specs=[
            pl.BlockSpec((NBLK, L, IN_C), lambda i: (i, 0, 0)),
            pl.BlockSpec((IN_C, HIDP), lambda i: (0, 0)),
            pl.BlockSpec((1, HIDP), lambda i: (0, 0)),
        ],
        out_specs=pl.BlockSpec((NBLK, HIDP), lambda i: (i, 0)),
        out_shape=jax.ShapeDtypeStruct((N, HIDP), f32),
    )(x, W1p, b1p)

    srcp, dstp, goff = pl.pallas_call(
        _edge_body,
        grid=(EB // EBLK,),
        in_specs=[pl.BlockSpec((2, EBLK, BATCH), lambda i: (0, i, 0))],
        out_specs=[
            pl.BlockSpec((EBLK, BATCH), lambda i: (i, 0)),
            pl.BlockSpec((EBLK, BATCH), lambda i: (i, 0)),
            pl.BlockSpec((2, EBLK, BATCH), lambda i: (0, i, 0)),
        ],
        out_shape=[
            jax.ShapeDtypeStruct((EB, BATCH), jnp.int32),
            jax.ShapeDtypeStruct((EB, BATCH), jnp.int32),
            jax.ShapeDtypeStruct((2, EB, BATCH), jnp.int32),
        ],
    )(ei3)

    srcp_r = srcp.reshape(2, 16, NB_DEG, BATCH)
    dstp_r1 = dstp.reshape(16, E_PAD // 16 // BG1, BG1)
    goff_r1 = goff.reshape(2, 16, E_PAD // 16 // BG1, BG1)
    dstp_r2 = dstp.reshape(16, E_PAD // 16 // BG2, BG2)
    goff_r2 = goff.reshape(2, 16, E_PAD // 16 // BG2, BG2)

    deg2 = _sc_deg(srcp_r, ones128, z640).reshape(2, NPAD_D, 1)

    hst1 = pl.pallas_call(
        _hs1_body,
        grid=(GRID_N,),
        in_specs=[
            pl.BlockSpec((2, NBLK, 1), lambda i: (0, i, 0)),
            pl.BlockSpec((NBLK, HIDP), lambda i: (i, 0)),
        ],
        out_specs=pl.BlockSpec((2, NBLK, T1), lambda i: (0, i, 0)),
        out_shape=jax.ShapeDtypeStruct((2, N, T1), f32),
    )(deg2, h1)

    acc1 = _sc_scat1(hst1.reshape(2 * N, T1), goff_r1, dstp_r1, z1)

    hst2 = pl.pallas_call(
        _mid_body,
        grid=(GRID_N,),
        in_specs=[
            pl.BlockSpec((2, NBLK, 1), lambda i: (0, i, 0)),
            pl.BlockSpec((2, NBLK, T1), lambda i: (0, i, 0)),
            pl.BlockSpec((2, NBLK, T1), lambda i: (0, i, 0)),
            pl.BlockSpec((HIDP, OUT_C), lambda i: (0, 0)),
            pl.BlockSpec((1, OUT_C), lambda i: (0, 0)),
        ],
        out_specs=pl.BlockSpec((2, NBLK, T2), lambda i: (0, i, 0)),
        out_shape=jax.ShapeDtypeStruct((2, N, T2), f32),
    )(deg2, acc1, hst1, W2p, b2r)

    acc2 = _sc_scat2(hst2.reshape(2 * N, T2), goff_r2, dstp_r2, z2)

    out = pl.pallas_call(
        _out_body,
        grid=(GRID_N,),
        in_specs=[
            pl.BlockSpec((2, NBLK, 1), lambda i: (0, i, 0)),
            pl.BlockSpec((2, NBLK, T2), lambda i: (0, i, 0)),
            pl.BlockSpec((2, NBLK, T2), lambda i: (0, i, 0)),
        ],
        out_specs=pl.BlockSpec((NBLK, OUT_C), lambda i: (i, 0)),
        out_shape=jax.ShapeDtypeStruct((N, OUT_C), f32),
    )(deg2, acc2, hst2)

    return out


# speedup vs baseline: 1.3761x; 1.0069x over previous
import functools

import jax
import jax.numpy as jnp
from jax import lax
from jax.experimental import pallas as pl
from jax.experimental.pallas import tpu as pltpu
from jax.experimental.pallas import tpu_sc as plsc

N = 10000
L = 4
IN_C = 128
HID = 300
HIDP = 304
OUT_C = 128
E = 320000
BATCH = 128
E_PAD = 327680
EB = 2560
NB_DEG = 80
BG = 64
NB_SCAT = 320
CHB = 64
NCH = NB_SCAT // CHB
NPAD_D = 10240
RPT_D = 640
NPAD_S = 10112
RPT_S = 632
TRASH = 10000
TRASH_ROWS = 112
T1 = HIDP // 2
T2 = OUT_C // 2
NBLK = 1000
GRID_N = N // NBLK

_HIGH = lax.Precision.HIGHEST


def _dinv(d_ref):
    return lax.rsqrt(d_ref[0, :, 0] + d_ref[1, :, 0] + 1.0)


def _h1_body(d_ref, x_ref, w_ref, b_ref, o_ref):
    dinv = _dinv(d_ref)
    xm = jnp.mean(x_ref[...], axis=1)
    h = (
        lax.dot_general(xm, w_ref[...], (((1,), (0,)), ((), ())),
                        precision=_HIGH)
        + b_ref[...]
    )
    hs = h * dinv[:, None]
    o_ref[0] = hs[:, :T1]
    o_ref[1] = hs[:, T1:]


EBLK = 128


def _edge_body(e_ref, srcp_ref, dstp_ref, goff_ref):
    i = pl.program_id(0)
    s = e_ref[0]
    d = e_ref[1]
    m = s == d
    base = (lax.broadcasted_iota(jnp.int32, (EBLK, BATCH), 0) * BATCH
            + lax.broadcasted_iota(jnp.int32, (EBLK, BATCH), 1)
            + i * (EBLK * BATCH))
    trash = TRASH + base % TRASH_ROWS
    srcp_ref[...] = jnp.where(m, trash, s)
    dstp_ref[...] = jnp.where(m, trash, d)
    g = jnp.where(m, (base * 9) % N, s)
    goff_ref[0] = g
    goff_ref[1] = g + N


def _mid_body(d_ref, acc_ref, hs_ref, w_ref, b_ref, o_ref):
    dinv = _dinv(d_ref)
    t = acc_ref[...] + hs_ref[...]
    p = jnp.concatenate([t[0], t[1]], axis=1) * dinv[:, None]
    p = jnp.where(p >= 0, p, 0.01 * p)
    h2 = (
        lax.dot_general(p, w_ref[...], (((1,), (0,)), ((), ())),
                        precision=_HIGH)
        + b_ref[...]
    )
    hs2 = h2 * dinv[:, None]
    o_ref[0] = hs2[:, :T2]
    o_ref[1] = hs2[:, T2:]


def _out_body(d_ref, acc_ref, hs_ref, o_ref):
    dinv = _dinv(d_ref)
    t = acc_ref[...] + hs_ref[...]
    o_ref[...] = jnp.concatenate([t[0], t[1]], axis=1) * dinv[:, None]


_MESH = plsc.VectorSubcoreMesh(core_axis_name="c", subcore_axis_name="s")
_SC_PARAMS = pltpu.CompilerParams(use_tc_tiling_on_sc=False)


@functools.partial(
    pl.kernel,
    mesh=_MESH,
    out_type=jax.ShapeDtypeStruct((2, NPAD_D), jnp.float32),
    compiler_params=_SC_PARAMS,
    scratch_types=[
        pltpu.VMEM((NB_DEG, BATCH), jnp.int32),
        pltpu.VMEM((BATCH,), jnp.float32),
        pltpu.VMEM((RPT_D,), jnp.float32),
        pltpu.VMEM_SHARED((NPAD_D,), jnp.float32),
    ],
)
def _sc_deg(srcp_hbm, ones_hbm, z640_hbm, deg_hbm, idxv, ones, obuf, degS):
    c = lax.axis_index("c")
    s = lax.axis_index("s")
    pltpu.sync_copy(ones_hbm, ones)
    pltpu.sync_copy(z640_hbm, obuf)
    pltpu.sync_copy(obuf, degS.at[pl.ds(s * RPT_D, RPT_D)])
    pltpu.sync_copy(srcp_hbm.at[c, s], idxv)
    plsc.subcore_barrier()

    @pl.loop(0, NB_DEG)
    def _(j):
        pltpu.sync_copy(ones, degS.at[idxv.at[j]], add=True)

    plsc.subcore_barrier()
    pltpu.sync_copy(degS.at[pl.ds(s * RPT_D, RPT_D)], obuf)
    pltpu.sync_copy(obuf, deg_hbm.at[c, pl.ds(s * RPT_D, RPT_D)])


def _make_sc_scat(T, bg, chb, dbuf):
    nb = E_PAD // 16 // bg
    nch = nb // chb
    assert nch % 2 == 0
    nfull = RPT_S // bg
    rem = RPT_S - nfull * bg
    scratch = [
        pltpu.VMEM((chb, bg), jnp.int32),
        pltpu.VMEM((chb, bg), jnp.int32),
        pltpu.VMEM((chb, bg), jnp.int32),
        pltpu.VMEM((chb, bg), jnp.int32),
        pltpu.VMEM((bg, T), jnp.float32),
        pltpu.VMEM((bg, T), jnp.float32),
        pltpu.VMEM_SHARED((NPAD_S, T), jnp.float32),
        pltpu.SemaphoreType.DMA,
        pltpu.SemaphoreType.DMA,
    ]

    def _body(hst_hbm, goff_hbm, dstp_hbm, zrows_hbm, acc_hbm,
              srcA, dstA, srcB, dstB, g0, g1, accS, sem, semi):
        c = lax.axis_index("c")
        s = lax.axis_index("s")
        pltpu.sync_copy(zrows_hbm, g0)

        @pl.loop(0, nfull)
        def _(k):
            pltpu.sync_copy(g0, accS.at[pl.ds(s * RPT_S + k * bg, bg)])

        if rem:
            pltpu.sync_copy(g0.at[pl.ds(0, rem)],
                            accS.at[pl.ds(s * RPT_S + nfull * bg, rem)])
        plsc.subcore_barrier()

        def _process(srcv, dstv):
            pltpu.async_copy(hst_hbm.at[srcv.at[0]], g0, sem)

            @pl.loop(0, chb // 2 - 1)
            def _(p):
                j = 2 * p
                pltpu.make_async_copy(hst_hbm.at[srcv.at[j]], g0, sem).wait()
                pltpu.async_copy(hst_hbm.at[srcv.at[j + 1]], g1, sem)
                pltpu.sync_copy(g0, accS.at[dstv.at[j]], add=True)
                pltpu.make_async_copy(
                    hst_hbm.at[srcv.at[j + 1]], g1, sem).wait()
                pltpu.async_copy(hst_hbm.at[srcv.at[j + 2]], g0, sem)
                pltpu.sync_copy(g1, accS.at[dstv.at[j + 1]], add=True)

            pltpu.make_async_copy(hst_hbm.at[srcv.at[chb - 2]], g0, sem).wait()
            pltpu.async_copy(hst_hbm.at[srcv.at[chb - 1]], g1, sem)
            pltpu.sync_copy(g0, accS.at[dstv.at[chb - 2]], add=True)
            pltpu.make_async_copy(hst_hbm.at[srcv.at[chb - 1]], g1, sem).wait()
            pltpu.sync_copy(g1, accS.at[dstv.at[chb - 1]], add=True)

        def _load_idx(q, sv, dv):
            pltpu.async_copy(goff_hbm.at[c, s, pl.ds(q * chb, chb)], sv, semi)
            pltpu.async_copy(dstp_hbm.at[s, pl.ds(q * chb, chb)], dv, semi)

        def _wait_idx(q, sv, dv):
            pltpu.make_async_copy(
                goff_hbm.at[c, s, pl.ds(q * chb, chb)], sv, semi).wait()
            pltpu.make_async_copy(
                dstp_hbm.at[s, pl.ds(q * chb, chb)], dv, semi).wait()

        pltpu.sync_copy(goff_hbm.at[c, s, pl.ds(0, chb)], srcA)
        pltpu.sync_copy(dstp_hbm.at[s, pl.ds(0, chb)], dstA)

        @pl.loop(0, nch // 2)
        def _(u):
            _load_idx(2 * u + 1, srcB, dstB)
            _process(srcA, dstA)
            _wait_idx(2 * u + 1, srcB, dstB)

            @pl.when(u < nch // 2 - 1)
            def _():
                _load_idx(2 * u + 2, srcA, dstA)

            _process(srcB, dstB)

            @pl.when(u < nch // 2 - 1)
            def _():
                _wait_idx(2 * u + 2, srcA, dstA)

        plsc.subcore_barrier()
        pltpu.sync_copy(accS.at[pl.ds(s * RPT_S, RPT_S)],
                        acc_hbm.at[c, pl.ds(s * RPT_S, RPT_S)])

    return pl.kernel(
        _body,
        mesh=_MESH,
        out_type=jax.ShapeDtypeStruct((2, NPAD_S, T), jnp.float32),
        compiler_params=_SC_PARAMS,
        scratch_types=scratch,
    )


BG1, CHB1 = 80, 32
BG2, CHB2 = 128, 80
_sc_scat1 = _make_sc_scat(T1, BG1, CHB1, True)
_sc_scat2 = _make_sc_scat(T2, BG2, CHB2, True)


def kernel(x, edge_index, W1, b1, W2, b2):
    f32 = jnp.float32
    W1p = jnp.pad(W1, ((0, 0), (0, HIDP - HID)))
    b1p = jnp.pad(b1, (0, HIDP - HID)).reshape(1, HIDP)
    W2p = jnp.pad(W2, ((0, HIDP - HID), (0, 0)))
    b2r = b2.reshape(1, OUT_C)
    ei3 = jnp.pad(edge_index, ((0, 0), (0, E_PAD - E))).reshape(2, EB, BATCH)

    ones128 = jnp.ones((BATCH,), f32)
    z640 = jnp.zeros((RPT_D,), f32)
    z1 = jnp.zeros((BG1, T1), f32)
    z2 = jnp.zeros((BG2, T2), f32)

    srcp, dstp, goff = pl.pallas_call(
        _edge_body,
        grid=(EB // EBLK,),
        in_specs=[pl.BlockSpec((2, EBLK, BATCH), lambda i: (0, i, 0))],
        out_specs=[
            pl.BlockSpec((EBLK, BATCH), lambda i: (i, 0)),
            pl.BlockSpec((EBLK, BATCH), lambda i: (i, 0)),
            pl.BlockSpec((2, EBLK, BATCH), lambda i: (0, i, 0)),
        ],
        out_shape=[
            jax.ShapeDtypeStruct((EB, BATCH), jnp.int32),
            jax.ShapeDtypeStruct((EB, BATCH), jnp.int32),
            jax.ShapeDtypeStruct((2, EB, BATCH), jnp.int32),
        ],
    )(ei3)

    srcp_r = srcp.reshape(2, 16, NB_DEG, BATCH)
    dstp_r1 = dstp.reshape(16, E_PAD // 16 // BG1, BG1)
    goff_r1 = goff.reshape(2, 16, E_PAD // 16 // BG1, BG1)
    dstp_r2 = dstp.reshape(16, E_PAD // 16 // BG2, BG2)
    goff_r2 = goff.reshape(2, 16, E_PAD // 16 // BG2, BG2)

    deg2 = _sc_deg(srcp_r, ones128, z640).reshape(2, NPAD_D, 1)

    hst1 = pl.pallas_call(
        _h1_body,
        grid=(GRID_N,),
        in_specs=[
            pl.BlockSpec((2, NBLK, 1), lambda i: (0, i, 0)),
            pl.BlockSpec((NBLK, L, IN_C), lambda i: (i, 0, 0)),
            pl.BlockSpec((IN_C, HIDP), lambda i: (0, 0)),
            pl.BlockSpec((1, HIDP), lambda i: (0, 0)),
        ],
        out_specs=pl.BlockSpec((2, NBLK, T1), lambda i: (0, i, 0)),
        out_shape=jax.ShapeDtypeStruct((2, N, T1), f32),
    )(deg2, x, W1p, b1p)

    acc1 = _sc_scat1(hst1.reshape(2 * N, T1), goff_r1, dstp_r1, z1)

    hst2 = pl.pallas_call(
        _mid_body,
        grid=(GRID_N,),
        in_specs=[
            pl.BlockSpec((2, NBLK, 1), lambda i: (0, i, 0)),
            pl.BlockSpec((2, NBLK, T1), lambda i: (0, i, 0)),
            pl.BlockSpec((2, NBLK, T1), lambda i: (0, i, 0)),
            pl.BlockSpec((HIDP, OUT_C), lambda i: (0, 0)),
            pl.BlockSpec((1, OUT_C), lambda i: (0, 0)),
        ],
        out_specs=pl.BlockSpec((2, NBLK, T2), lambda i: (0, i, 0)),
        out_shape=jax.ShapeDtypeStruct((2, N, T2), f32),
    )(deg2, acc1, hst1, W2p, b2r)

    acc2 = _sc_scat2(hst2.reshape(2 * N, T2), goff_r2, dstp_r2, z2)

    out = pl.pallas_call(
        _out_body,
        grid=(GRID_N,),
        in_specs=[
            pl.BlockSpec((2, NBLK, 1), lambda i: (0, i, 0)),
            pl.BlockSpec((2, NBLK, T2), lambda i: (0, i, 0)),
            pl.BlockSpec((2, NBLK, T2), lambda i: (0, i, 0)),
        ],
        out_specs=pl.BlockSpec((NBLK, OUT_C), lambda i: (i, 0)),
        out_shape=jax.ShapeDtypeStruct((N, OUT_C), f32),
    )(deg2, acc2, hst2)

    return out
